# baseline (speedup 1.0000x reference)
import functools

import numpy as np
import jax
import jax.numpy as jnp
from jax import lax
from jax.experimental import pallas as pl
from jax.experimental.pallas import tpu as pltpu
from jax.experimental.pallas import tpu_sc as plsc

_EMBED_DIM = 64
_BATCH = 16384
_WSIZE = 2_000_000
_NC, _NS = 2, 16
_NW = _NC * _NS
_ROWS = _BATCH // _NW
_GROUPS = _ROWS // 16
_CHUNK = 2048
_NCHUNK = _ROWS * _EMBED_DIM // _CHUNK
_FIRE = 16
_BLKGRP = 8
_CPB = _BLKGRP * 16 * _EMBED_DIM // _CHUNK

_rs = np.random.RandomState(1024)
_rn = np.concatenate(
    [np.array([2038074743]), _rs.randint(0, 2038074743, (10,))]
).astype(np.int64)
_P = int(_rn[0])
_A = int(_rn[1])
_B = int(_rn[2])
_R = int(_rn[3])

_THI = np.array([(h * 1024 * _R + _A) % _P for h in range(1024)], dtype=np.int32)
_TLO = np.array([(l * _R) % _P for l in range(1024)], dtype=np.int32)
_OFF = np.array([(j * _B) % _P for j in range(_EMBED_DIM)], dtype=np.int32)
_PMO = np.array([_P - int(o) for o in _OFF], dtype=np.int32)
_INV_W = np.float32(1.0 / _WSIZE)

_mesh = plsc.VectorSubcoreMesh(core_axis_name="c", subcore_axis_name="s")


@functools.partial(
    pl.kernel,
    out_type=jax.ShapeDtypeStruct((_BATCH, _EMBED_DIM), jnp.float32),
    mesh=_mesh,
    scratch_types=[
        pltpu.VMEM((_ROWS,), jnp.int32),
        pltpu.VMEM((1024,), jnp.int32),
        pltpu.VMEM((1024,), jnp.int32),
        pltpu.VMEM((_ROWS,), jnp.int32),
        pltpu.VMEM((_EMBED_DIM,), jnp.int32),
        pltpu.VMEM((_EMBED_DIM,), jnp.int32),
        pltpu.VMEM((_ROWS * _EMBED_DIM,), jnp.int32),
        pltpu.VMEM((_ROWS, _EMBED_DIM), jnp.float32),
        pltpu.SemaphoreType.DMA,
    ],
    compiler_params=pltpu.CompilerParams(needs_layout_passes=False, use_tc_tiling_on_sc=True),
)
def _hashed_lookup(table_h, idx_h, thi_h, tlo_h, off_h, pmo_h, out_h,
                   idx_v, thi_v, tlo_v, base_v, off_v, pmo_v, hidx_v,
                   gath_v, sem):
    wid = lax.axis_index("s") * jnp.int32(_NC) + lax.axis_index("c")
    row0 = wid * jnp.int32(_ROWS)
    pltpu.sync_copy(idx_h.at[pl.ds(row0, _ROWS)], idx_v)
    pltpu.sync_copy(thi_h, thi_v)
    pltpu.sync_copy(tlo_h, tlo_v)
    pltpu.sync_copy(off_h, off_v)
    pltpu.sync_copy(pmo_h, pmo_v)

    pconst = jnp.int32(_P)
    wconst = jnp.int32(_WSIZE)

    def phase_a(k, carry):
        o = k * jnp.int32(16)
        v = idx_v[pl.ds(o, 16)]
        hi = lax.shift_right_logical(v, jnp.int32(10))
        lo = lax.bitwise_and(v, jnp.int32(1023))
        a = plsc.load_gather(thi_v, [hi])
        b = plsc.load_gather(tlo_v, [lo])
        d = a - (pconst - b)
        base_v[pl.ds(o, 16)] = jnp.where(d >= 0, d, a + b)
        return carry

    lax.fori_loop(jnp.int32(0), jnp.int32(_GROUPS), phase_a, jnp.int32(0))

    offs = [off_v[pl.ds(k * 16, 16)] for k in range(_EMBED_DIM // 16)]
    pmos = [pmo_v[pl.ds(k * 16, 16)] for k in range(_EMBED_DIM // 16)]

    def phase_b(g, carry):
        gbase = g * jnp.int32(16 * _EMBED_DIM)
        gsel = g * jnp.int32(16)
        for r in range(16):
            sel = gsel + jnp.full((16,), r, dtype=jnp.int32)
            bi = plsc.load_gather(base_v, [sel])
            for k in range(_EMBED_DIM // 16):
                d = bi - pmos[k]
                t = jnp.where(d >= 0, d, bi + offs[k])
                q = (t.astype(jnp.float32) * _INV_W).astype(jnp.int32)
                rem = t - q * wconst
                rem = jnp.where(rem < 0, rem + wconst, rem)
                rem = jnp.where(rem >= wconst, rem - wconst, rem)
                hidx_v[pl.ds(gbase + jnp.int32(r * _EMBED_DIM + k * 16), 16)] = rem
        return carry

    def block_fn(blk, carry):
        g0 = blk * jnp.int32(_BLKGRP)

        def grp(g2, c2):
            return phase_b(g0 + g2, c2)

        lax.fori_loop(jnp.int32(0), jnp.int32(_BLKGRP), grp, carry)
        cr0 = blk * jnp.int32(_BLKGRP * 16)

        def fire(rr, c2):
            pltpu.async_copy(
                table_h.at[hidx_v.at[pl.ds(rr * jnp.int32(_EMBED_DIM),
                                           _EMBED_DIM)]],
                gath_v.at[rr],
                sem,
            )
            return c2

        lax.fori_loop(cr0, cr0 + jnp.int32(_BLKGRP * 16), fire, carry)
        return carry

    lax.fori_loop(jnp.int32(0), jnp.int32(_GROUPS // _BLKGRP), block_fn,
                  jnp.int32(0))

    def drain(rr, c2):
        pltpu.make_async_copy(
            table_h.at[pl.ds(0, _EMBED_DIM)],
            gath_v.at[rr],
            sem,
        ).wait()
        return c2

    lax.fori_loop(jnp.int32(0), jnp.int32(_ROWS), drain, jnp.int32(0))

    pltpu.sync_copy(gath_v, out_h.at[pl.ds(row0, _ROWS)])


def kernel(hashed_weights, indices):
    idx32 = indices.astype(jnp.int32)
    out = _hashed_lookup(
        hashed_weights, idx32,
        jnp.asarray(_THI), jnp.asarray(_TLO),
        jnp.asarray(_OFF), jnp.asarray(_PMO),
    )
    return out


# speedup vs baseline: 1.0826x; 1.0826x over previous
import functools

import numpy as np
import jax
import jax.numpy as jnp
from jax import lax
from jax.experimental import pallas as pl
from jax.experimental.pallas import tpu as pltpu
from jax.experimental.pallas import tpu_sc as plsc

_EMBED_DIM = 64
_BATCH = 16384
_WSIZE = 2_000_000
_NC, _NS = 2, 16
_NW = _NC * _NS
_ROWS = _BATCH // _NW
_GROUPS = _ROWS // 16
_CHUNK = 2048
_NCHUNK = _ROWS * _EMBED_DIM // _CHUNK
_FIRE = 16
_BLKGRP = 8
_CPB = _BLKGRP * 16 * _EMBED_DIM // _CHUNK

_rs = np.random.RandomState(1024)
_rn = np.concatenate(
    [np.array([2038074743]), _rs.randint(0, 2038074743, (10,))]
).astype(np.int64)
_P = int(_rn[0])
_A = int(_rn[1])
_B = int(_rn[2])
_R = int(_rn[3])

_C1024R = (1024 * _R) % _P
_INV_W = np.float32(1.0 / _WSIZE)

_mesh = plsc.VectorSubcoreMesh(core_axis_name="c", subcore_axis_name="s")


@functools.partial(
    pl.kernel,
    out_type=jax.ShapeDtypeStruct((_BATCH, _EMBED_DIM), jnp.float32),
    mesh=_mesh,
    scratch_types=[
        pltpu.VMEM((_ROWS,), jnp.int32),
        pltpu.VMEM((1024,), jnp.int32),
        pltpu.VMEM((1024,), jnp.int32),
        pltpu.VMEM((_ROWS,), jnp.int32),
        pltpu.VMEM((_EMBED_DIM,), jnp.int32),
        pltpu.VMEM((_ROWS * _EMBED_DIM,), jnp.int32),
        pltpu.VMEM((_ROWS, _EMBED_DIM), jnp.float32),
        pltpu.SemaphoreType.DMA,
    ],
    compiler_params=pltpu.CompilerParams(needs_layout_passes=False),
)
def _hashed_lookup(table_h, idx_h, out_h,
                   idx_v, thi_v, tlo_v, base_v, off_v, hidx_v,
                   gath_v, sem):
    wid = lax.axis_index("s") * jnp.int32(_NC) + lax.axis_index("c")
    row0 = wid * jnp.int32(_ROWS)
    pltpu.sync_copy(idx_h.at[pl.ds(row0, _ROWS)], idx_v)

    pconst = jnp.int32(_P)
    wconst = jnp.int32(_WSIZE)
    iota = lax.iota(jnp.int32, 16)
    pmi = pconst - iota

    def _mulmod_iota(c):
        res = jnp.zeros((16,), jnp.int32)
        for bit in bin(c)[2:]:
            t = res - (pconst - res)
            res = jnp.where(t >= 0, t, res + res)
            if bit == "1":
                t = res - pmi
                res = jnp.where(t >= 0, t, res + iota)
        return res

    def _fill(ref, n_groups, v0, step):
        pms = jnp.int32(_P - step)
        sconst = jnp.int32(step)

        def body(g, v):
            ref[pl.ds(g * jnp.int32(16), 16)] = v
            t = v - pms
            return jnp.where(t >= 0, t, v + sconst)

        lax.fori_loop(jnp.int32(0), jnp.int32(n_groups), body, v0)

    v_hi = _mulmod_iota(_C1024R)
    t0 = v_hi - jnp.int32(_P - _A)
    v_hi = jnp.where(t0 >= 0, t0, v_hi + jnp.int32(_A))
    _fill(thi_v, 64, v_hi, (16 * _C1024R) % _P)
    _fill(tlo_v, 64, _mulmod_iota(_R), (16 * _R) % _P)
    _fill(off_v, _EMBED_DIM // 16, _mulmod_iota(_B), (16 * _B) % _P)

    def phase_a(k, carry):
        o = k * jnp.int32(16)
        v = idx_v[pl.ds(o, 16)]
        hi = lax.shift_right_logical(v, jnp.int32(10))
        lo = lax.bitwise_and(v, jnp.int32(1023))
        a = plsc.load_gather(thi_v, [hi])
        b = plsc.load_gather(tlo_v, [lo])
        d = a - (pconst - b)
        base_v[pl.ds(o, 16)] = jnp.where(d >= 0, d, a + b)
        return carry

    lax.fori_loop(jnp.int32(0), jnp.int32(_GROUPS), phase_a, jnp.int32(0))

    offs = [off_v[pl.ds(k * 16, 16)] for k in range(_EMBED_DIM // 16)]
    pmos = [pconst - o for o in offs]

    def phase_b(g, carry):
        gbase = g * jnp.int32(16 * _EMBED_DIM)
        gsel = g * jnp.int32(16)
        for r in range(16):
            sel = gsel + jnp.full((16,), r, dtype=jnp.int32)
            bi = plsc.load_gather(base_v, [sel])
            for k in range(_EMBED_DIM // 16):
                d = bi - pmos[k]
                t = jnp.where(d >= 0, d, bi + offs[k])
                q = (t.astype(jnp.float32) * _INV_W).astype(jnp.int32)
                rem = t - q * wconst
                rem = jnp.where(rem < 0, rem + wconst, rem)
                rem = jnp.where(rem >= wconst, rem - wconst, rem)
                hidx_v[pl.ds(gbase + jnp.int32(r * _EMBED_DIM + k * 16), 16)] = rem
        return carry

    def block_fn(blk, carry):
        g0 = blk * jnp.int32(_BLKGRP)

        def grp(g2, c2):
            return phase_b(g0 + g2, c2)

        lax.fori_loop(jnp.int32(0), jnp.int32(_BLKGRP), grp, carry)
        cr0 = blk * jnp.int32(_BLKGRP * 16)

        def fire(rr, c2):
            pltpu.async_copy(
                table_h.at[hidx_v.at[pl.ds(rr * jnp.int32(_EMBED_DIM),
                                           _EMBED_DIM)]],
                gath_v.at[rr],
                sem,
            )
            return c2

        lax.fori_loop(cr0, cr0 + jnp.int32(_BLKGRP * 16), fire, carry)
        return carry

    lax.fori_loop(jnp.int32(0), jnp.int32(_GROUPS // _BLKGRP), block_fn,
                  jnp.int32(0))

    def drain(rr, c2):
        pltpu.make_async_copy(
            table_h.at[pl.ds(0, _EMBED_DIM)],
            gath_v.at[rr],
            sem,
        ).wait()
        return c2

    lax.fori_loop(jnp.int32(0), jnp.int32(_ROWS), drain, jnp.int32(0))

    pltpu.sync_copy(gath_v, out_h.at[pl.ds(row0, _ROWS)])


def kernel(hashed_weights, indices):
    idx32 = indices.astype(jnp.int32)
    return _hashed_lookup(hashed_weights, idx32)


# speedup vs baseline: 1.1032x; 1.0190x over previous
import functools

import numpy as np
import jax
import jax.numpy as jnp
from jax import lax
from jax.experimental import pallas as pl
from jax.experimental.pallas import tpu as pltpu
from jax.experimental.pallas import tpu_sc as plsc

_EMBED_DIM = 64
_BATCH = 16384
_WSIZE = 2_000_000
_NC, _NS = 2, 16
_NW = _NC * _NS
_ROWS = _BATCH // _NW
_GROUPS = _ROWS // 16
_CHUNK = 2048
_NCHUNK = _ROWS * _EMBED_DIM // _CHUNK
_FIRE = 16
_BLKGRP = 4
_CPB = _BLKGRP * 16 * _EMBED_DIM // _CHUNK

_rs = np.random.RandomState(1024)
_rn = np.concatenate(
    [np.array([2038074743]), _rs.randint(0, 2038074743, (10,))]
).astype(np.int64)
_P = int(_rn[0])
_A = int(_rn[1])
_B = int(_rn[2])
_R = int(_rn[3])

_C1024R = (1024 * _R) % _P
_INV_W = np.float32(1.0 / _WSIZE)

_mesh = plsc.VectorSubcoreMesh(core_axis_name="c", subcore_axis_name="s")


@functools.partial(
    pl.kernel,
    out_type=jax.ShapeDtypeStruct((_BATCH, _EMBED_DIM), jnp.float32),
    mesh=_mesh,
    scratch_types=[
        pltpu.VMEM((_ROWS,), jnp.int32),
        pltpu.VMEM((1024,), jnp.int32),
        pltpu.VMEM((1024,), jnp.int32),
        pltpu.VMEM((_ROWS,), jnp.int32),
        pltpu.VMEM((_EMBED_DIM,), jnp.int32),
        pltpu.VMEM((_ROWS * _EMBED_DIM,), jnp.int32),
        pltpu.VMEM((_ROWS, _EMBED_DIM), jnp.float32),
        pltpu.SemaphoreType.DMA,
    ],
    compiler_params=pltpu.CompilerParams(needs_layout_passes=False),
)
def _hashed_lookup(table_h, idx_h, out_h,
                   idx_v, thi_v, tlo_v, base_v, off_v, hidx_v,
                   gath_v, sem):
    wid = lax.axis_index("s") * jnp.int32(_NC) + lax.axis_index("c")
    row0 = wid * jnp.int32(_ROWS)
    pltpu.sync_copy(idx_h.at[pl.ds(row0, _ROWS)], idx_v)

    pconst = jnp.int32(_P)
    wconst = jnp.int32(_WSIZE)
    iota = lax.iota(jnp.int32, 16)
    pmi = pconst - iota

    def _mulmod_iota(c):
        res = jnp.zeros((16,), jnp.int32)
        for bit in bin(c)[2:]:
            t = res - (pconst - res)
            res = jnp.where(t >= 0, t, res + res)
            if bit == "1":
                t = res - pmi
                res = jnp.where(t >= 0, t, res + iota)
        return res

    def _fill(ref, n_groups, v0, step):
        pms = jnp.int32(_P - step)
        sconst = jnp.int32(step)

        def body(g, v):
            ref[pl.ds(g * jnp.int32(16), 16)] = v
            t = v - pms
            return jnp.where(t >= 0, t, v + sconst)

        lax.fori_loop(jnp.int32(0), jnp.int32(n_groups), body, v0)

    v_hi = _mulmod_iota(_C1024R)
    t0 = v_hi - jnp.int32(_P - _A)
    v_hi = jnp.where(t0 >= 0, t0, v_hi + jnp.int32(_A))
    _fill(thi_v, 64, v_hi, (16 * _C1024R) % _P)
    _fill(tlo_v, 64, _mulmod_iota(_R), (16 * _R) % _P)
    _fill(off_v, _EMBED_DIM // 16, _mulmod_iota(_B), (16 * _B) % _P)

    def phase_a(k, carry):
        o = k * jnp.int32(16)
        v = idx_v[pl.ds(o, 16)]
        hi = lax.shift_right_logical(v, jnp.int32(10))
        lo = lax.bitwise_and(v, jnp.int32(1023))
        a = plsc.load_gather(thi_v, [hi])
        b = plsc.load_gather(tlo_v, [lo])
        d = a - (pconst - b)
        base_v[pl.ds(o, 16)] = jnp.where(d >= 0, d, a + b)
        return carry

    lax.fori_loop(jnp.int32(0), jnp.int32(_GROUPS), phase_a, jnp.int32(0))

    offs = [off_v[pl.ds(k * 16, 16)] for k in range(_EMBED_DIM // 16)]
    pmos = [pconst - o for o in offs]

    def phase_b(g, carry):
        gbase = g * jnp.int32(16 * _EMBED_DIM)
        gsel = g * jnp.int32(16)
        for r in range(16):
            sel = gsel + jnp.full((16,), r, dtype=jnp.int32)
            bi = plsc.load_gather(base_v, [sel])
            for k in range(_EMBED_DIM // 16):
                d = bi - pmos[k]
                t = jnp.where(d >= 0, d, bi + offs[k])
                q = (t.astype(jnp.float32) * _INV_W).astype(jnp.int32)
                rem = t - q * wconst
                rem = jnp.where(rem < 0, rem + wconst, rem)
                rem = jnp.where(rem >= wconst, rem - wconst, rem)
                hidx_v[pl.ds(gbase + jnp.int32(r * _EMBED_DIM + k * 16), 16)] = rem
        return carry

    def block_fn(blk, carry):
        g0 = blk * jnp.int32(_BLKGRP)

        def grp(g2, c2):
            return phase_b(g0 + g2, c2)

        lax.fori_loop(jnp.int32(0), jnp.int32(_BLKGRP), grp, carry)
        cr0 = blk * jnp.int32(_BLKGRP * 16)

        def fire(rr, c2):
            pltpu.async_copy(
                table_h.at[hidx_v.at[pl.ds(rr * jnp.int32(_EMBED_DIM),
                                           _EMBED_DIM)]],
                gath_v.at[rr],
                sem,
            )
            return c2

        lax.fori_loop(cr0, cr0 + jnp.int32(_BLKGRP * 16), fire, carry)
        return carry

    lax.fori_loop(jnp.int32(0), jnp.int32(_GROUPS // _BLKGRP), block_fn,
                  jnp.int32(0))

    def drain(rr, c2):
        pltpu.make_async_copy(
            table_h.at[pl.ds(0, _EMBED_DIM)],
            gath_v.at[rr],
            sem,
        ).wait()
        return c2

    lax.fori_loop(jnp.int32(0), jnp.int32(_ROWS), drain, jnp.int32(0))

    pltpu.sync_copy(gath_v, out_h.at[pl.ds(row0, _ROWS)])


def kernel(hashed_weights, indices):
    idx32 = indices.astype(jnp.int32)
    return _hashed_lookup(hashed_weights, idx32)


# speedup vs baseline: 1.1062x; 1.0028x over previous
import functools

import numpy as np
import jax
import jax.numpy as jnp
from jax import lax
from jax.experimental import pallas as pl
from jax.experimental.pallas import tpu as pltpu
from jax.experimental.pallas import tpu_sc as plsc

_EMBED_DIM = 64
_BATCH = 16384
_WSIZE = 2_000_000
_NC, _NS = 2, 16
_NW = _NC * _NS
_ROWS = _BATCH // _NW
_GROUPS = _ROWS // 16
_CHUNK = 2048
_NCHUNK = _ROWS * _EMBED_DIM // _CHUNK
_FIRE = 16
_BLKGRP = 2
_CPB = _BLKGRP * 16 * _EMBED_DIM // _CHUNK

_rs = np.random.RandomState(1024)
_rn = np.concatenate(
    [np.array([2038074743]), _rs.randint(0, 2038074743, (10,))]
).astype(np.int64)
_P = int(_rn[0])
_A = int(_rn[1])
_B = int(_rn[2])
_R = int(_rn[3])

_C1024R = (1024 * _R) % _P
_INV_W = np.float32(1.0 / _WSIZE)

_mesh = plsc.VectorSubcoreMesh(core_axis_name="c", subcore_axis_name="s")


@functools.partial(
    pl.kernel,
    out_type=jax.ShapeDtypeStruct((_BATCH, _EMBED_DIM), jnp.float32),
    mesh=_mesh,
    scratch_types=[
        pltpu.VMEM((_ROWS,), jnp.int32),
        pltpu.VMEM((1024,), jnp.int32),
        pltpu.VMEM((1024,), jnp.int32),
        pltpu.VMEM((_ROWS,), jnp.int32),
        pltpu.VMEM((_EMBED_DIM,), jnp.int32),
        pltpu.VMEM((_ROWS * _EMBED_DIM,), jnp.int32),
        pltpu.VMEM((_ROWS, _EMBED_DIM), jnp.float32),
        pltpu.SemaphoreType.DMA,
    ],
    compiler_params=pltpu.CompilerParams(needs_layout_passes=False),
)
def _hashed_lookup(table_h, idx_h, out_h,
                   idx_v, thi_v, tlo_v, base_v, off_v, hidx_v,
                   gath_v, sem):
    wid = lax.axis_index("s") * jnp.int32(_NC) + lax.axis_index("c")
    row0 = wid * jnp.int32(_ROWS)
    pltpu.sync_copy(idx_h.at[pl.ds(row0, _ROWS)], idx_v)

    pconst = jnp.int32(_P)
    wconst = jnp.int32(_WSIZE)
    iota = lax.iota(jnp.int32, 16)
    pmi = pconst - iota

    def _mulmod_iota(c):
        res = jnp.zeros((16,), jnp.int32)
        for bit in bin(c)[2:]:
            t = res - (pconst - res)
            res = jnp.where(t >= 0, t, res + res)
            if bit == "1":
                t = res - pmi
                res = jnp.where(t >= 0, t, res + iota)
        return res

    def _fill(ref, n_groups, v0, step):
        pms = jnp.int32(_P - step)
        sconst = jnp.int32(step)

        def body(g, v):
            ref[pl.ds(g * jnp.int32(16), 16)] = v
            t = v - pms
            return jnp.where(t >= 0, t, v + sconst)

        lax.fori_loop(jnp.int32(0), jnp.int32(n_groups), body, v0)

    v_hi = _mulmod_iota(_C1024R)
    t0 = v_hi - jnp.int32(_P - _A)
    v_hi = jnp.where(t0 >= 0, t0, v_hi + jnp.int32(_A))
    _fill(thi_v, 64, v_hi, (16 * _C1024R) % _P)
    _fill(tlo_v, 64, _mulmod_iota(_R), (16 * _R) % _P)
    _fill(off_v, _EMBED_DIM // 16, _mulmod_iota(_B), (16 * _B) % _P)

    def phase_a(k, carry):
        o = k * jnp.int32(16)
        v = idx_v[pl.ds(o, 16)]
        hi = lax.shift_right_logical(v, jnp.int32(10))
        lo = lax.bitwise_and(v, jnp.int32(1023))
        a = plsc.load_gather(thi_v, [hi])
        b = plsc.load_gather(tlo_v, [lo])
        d = a - (pconst - b)
        base_v[pl.ds(o, 16)] = jnp.where(d >= 0, d, a + b)
        return carry

    lax.fori_loop(jnp.int32(0), jnp.int32(_GROUPS), phase_a, jnp.int32(0))

    offs = [off_v[pl.ds(k * 16, 16)] for k in range(_EMBED_DIM // 16)]
    pmos = [pconst - o for o in offs]

    def phase_b(g, carry):
        gbase = g * jnp.int32(16 * _EMBED_DIM)
        gsel = g * jnp.int32(16)
        for r in range(16):
            sel = gsel + jnp.full((16,), r, dtype=jnp.int32)
            bi = plsc.load_gather(base_v, [sel])
            for k in range(_EMBED_DIM // 16):
                d = bi - pmos[k]
                t = jnp.where(d >= 0, d, bi + offs[k])
                q = (t.astype(jnp.float32) * _INV_W).astype(jnp.int32)
                rem = t - q * wconst
                rem = jnp.where(rem < 0, rem + wconst, rem)
                rem = jnp.where(rem >= wconst, rem - wconst, rem)
                hidx_v[pl.ds(gbase + jnp.int32(r * _EMBED_DIM + k * 16), 16)] = rem
        return carry

    def block_fn(blk, carry):
        g0 = blk * jnp.int32(_BLKGRP)

        def grp(g2, c2):
            return phase_b(g0 + g2, c2)

        lax.fori_loop(jnp.int32(0), jnp.int32(_BLKGRP), grp, carry)
        cr0 = blk * jnp.int32(_BLKGRP * 16)

        def fire(rr, c2):
            pltpu.async_copy(
                table_h.at[hidx_v.at[pl.ds(rr * jnp.int32(_EMBED_DIM),
                                           _EMBED_DIM)]],
                gath_v.at[rr],
                sem,
            )
            return c2

        lax.fori_loop(cr0, cr0 + jnp.int32(_BLKGRP * 16), fire, carry)
        return carry

    lax.fori_loop(jnp.int32(0), jnp.int32(_GROUPS // _BLKGRP), block_fn,
                  jnp.int32(0))

    def drain(rr, c2):
        pltpu.make_async_copy(
            table_h.at[pl.ds(0, _EMBED_DIM)],
            gath_v.at[rr],
            sem,
        ).wait()
        return c2

    lax.fori_loop(jnp.int32(0), jnp.int32(_ROWS), drain, jnp.int32(0))

    pltpu.sync_copy(gath_v, out_h.at[pl.ds(row0, _ROWS)])


def kernel(hashed_weights, indices):
    idx32 = indices.astype(jnp.int32)
    return _hashed_lookup(hashed_weights, idx32)


# speedup vs baseline: 1.1152x; 1.0081x over previous
import functools

import numpy as np
import jax
import jax.numpy as jnp
from jax import lax
from jax.experimental import pallas as pl
from jax.experimental.pallas import tpu as pltpu
from jax.experimental.pallas import tpu_sc as plsc

_EMBED_DIM = 64
_BATCH = 16384
_WSIZE = 2_000_000
_NC, _NS = 2, 16
_NW = _NC * _NS
_ROWS = _BATCH // _NW
_GROUPS = _ROWS // 16
_CHUNK = 2048
_NCHUNK = _ROWS * _EMBED_DIM // _CHUNK
_FIRE = 16
_BLKGRP = 1
_CPB = _BLKGRP * 16 * _EMBED_DIM // _CHUNK

_rs = np.random.RandomState(1024)
_rn = np.concatenate(
    [np.array([2038074743]), _rs.randint(0, 2038074743, (10,))]
).astype(np.int64)
_P = int(_rn[0])
_A = int(_rn[1])
_B = int(_rn[2])
_R = int(_rn[3])

_C1024R = (1024 * _R) % _P
_INV_W = np.float32(1.0 / _WSIZE)

_mesh = plsc.VectorSubcoreMesh(core_axis_name="c", subcore_axis_name="s")


@functools.partial(
    pl.kernel,
    out_type=jax.ShapeDtypeStruct((_BATCH, _EMBED_DIM), jnp.float32),
    mesh=_mesh,
    scratch_types=[
        pltpu.VMEM((_ROWS,), jnp.int32),
        pltpu.VMEM((1024,), jnp.int32),
        pltpu.VMEM((1024,), jnp.int32),
        pltpu.VMEM((_ROWS,), jnp.int32),
        pltpu.VMEM((_EMBED_DIM,), jnp.int32),
        pltpu.VMEM((_ROWS * _EMBED_DIM,), jnp.int32),
        pltpu.VMEM((_ROWS, _EMBED_DIM), jnp.float32),
        pltpu.SemaphoreType.DMA,
    ],
    compiler_params=pltpu.CompilerParams(needs_layout_passes=False),
)
def _hashed_lookup(table_h, idx_h, out_h,
                   idx_v, thi_v, tlo_v, base_v, off_v, hidx_v,
                   gath_v, sem):
    wid = lax.axis_index("s") * jnp.int32(_NC) + lax.axis_index("c")
    row0 = wid * jnp.int32(_ROWS)
    pltpu.sync_copy(idx_h.at[pl.ds(row0, _ROWS)], idx_v)

    pconst = jnp.int32(_P)
    wconst = jnp.int32(_WSIZE)
    iota = lax.iota(jnp.int32, 16)
    pmi = pconst - iota

    def _mulmod_iota(c):
        res = jnp.zeros((16,), jnp.int32)
        for bit in bin(c)[2:]:
            t = res - (pconst - res)
            res = jnp.where(t >= 0, t, res + res)
            if bit == "1":
                t = res - pmi
                res = jnp.where(t >= 0, t, res + iota)
        return res

    def _fill(ref, n_groups, v0, step):
        pms = jnp.int32(_P - step)
        sconst = jnp.int32(step)

        def body(g, v):
            ref[pl.ds(g * jnp.int32(16), 16)] = v
            t = v - pms
            return jnp.where(t >= 0, t, v + sconst)

        lax.fori_loop(jnp.int32(0), jnp.int32(n_groups), body, v0)

    v_hi = _mulmod_iota(_C1024R)
    t0 = v_hi - jnp.int32(_P - _A)
    v_hi = jnp.where(t0 >= 0, t0, v_hi + jnp.int32(_A))
    _fill(thi_v, 64, v_hi, (16 * _C1024R) % _P)
    _fill(tlo_v, 64, _mulmod_iota(_R), (16 * _R) % _P)
    _fill(off_v, _EMBED_DIM // 16, _mulmod_iota(_B), (16 * _B) % _P)

    def phase_a(k, carry):
        o = k * jnp.int32(16)
        v = idx_v[pl.ds(o, 16)]
        hi = lax.shift_right_logical(v, jnp.int32(10))
        lo = lax.bitwise_and(v, jnp.int32(1023))
        a = plsc.load_gather(thi_v, [hi])
        b = plsc.load_gather(tlo_v, [lo])
        d = a - (pconst - b)
        base_v[pl.ds(o, 16)] = jnp.where(d >= 0, d, a + b)
        return carry

    lax.fori_loop(jnp.int32(0), jnp.int32(_GROUPS), phase_a, jnp.int32(0))

    offs = [off_v[pl.ds(k * 16, 16)] for k in range(_EMBED_DIM // 16)]
    pmos = [pconst - o for o in offs]

    def phase_b(g, carry):
        gbase = g * jnp.int32(16 * _EMBED_DIM)
        gsel = g * jnp.int32(16)
        for r in range(16):
            sel = gsel + jnp.full((16,), r, dtype=jnp.int32)
            bi = plsc.load_gather(base_v, [sel])
            for k in range(_EMBED_DIM // 16):
                d = bi - pmos[k]
                t = jnp.where(d >= 0, d, bi + offs[k])
                q = (t.astype(jnp.float32) * _INV_W).astype(jnp.int32)
                rem = t - q * wconst
                rem = jnp.where(rem < 0, rem + wconst, rem)
                rem = jnp.where(rem >= wconst, rem - wconst, rem)
                hidx_v[pl.ds(gbase + jnp.int32(r * _EMBED_DIM + k * 16), 16)] = rem
        return carry

    def block_fn(blk, carry):
        g0 = blk * jnp.int32(_BLKGRP)

        def grp(g2, c2):
            return phase_b(g0 + g2, c2)

        lax.fori_loop(jnp.int32(0), jnp.int32(_BLKGRP), grp, carry)
        cr0 = blk * jnp.int32(_BLKGRP * 16)

        def fire(rr, c2):
            pltpu.async_copy(
                table_h.at[hidx_v.at[pl.ds(rr * jnp.int32(_EMBED_DIM),
                                           _EMBED_DIM)]],
                gath_v.at[rr],
                sem,
            )
            return c2

        lax.fori_loop(cr0, cr0 + jnp.int32(_BLKGRP * 16), fire, carry)
        return carry

    lax.fori_loop(jnp.int32(0), jnp.int32(_GROUPS // _BLKGRP), block_fn,
                  jnp.int32(0))

    def drain(rr, c2):
        pltpu.make_async_copy(
            table_h.at[pl.ds(0, _EMBED_DIM)],
            gath_v.at[rr],
            sem,
        ).wait()
        return c2

    lax.fori_loop(jnp.int32(0), jnp.int32(_ROWS), drain, jnp.int32(0))

    pltpu.sync_copy(gath_v, out_h.at[pl.ds(row0, _ROWS)])


def kernel(hashed_weights, indices):
    idx32 = indices.astype(jnp.int32)
    return _hashed_lookup(hashed_weights, idx32)


# speedup vs baseline: 1.1185x; 1.0029x over previous
import functools

import numpy as np
import jax
import jax.numpy as jnp
from jax import lax
from jax.experimental import pallas as pl
from jax.experimental.pallas import tpu as pltpu
from jax.experimental.pallas import tpu_sc as plsc

_EMBED_DIM = 64
_BATCH = 16384
_WSIZE = 2_000_000
_NC, _NS = 2, 16
_NW = _NC * _NS
_ROWS = _BATCH // _NW
_GROUPS = _ROWS // 16
_CHUNK = 2048
_NCHUNK = _ROWS * _EMBED_DIM // _CHUNK
_FIRE = 16
_BLKGRP = 1
_CPB = _BLKGRP * 16 * _EMBED_DIM // _CHUNK

_rs = np.random.RandomState(1024)
_rn = np.concatenate(
    [np.array([2038074743]), _rs.randint(0, 2038074743, (10,))]
).astype(np.int64)
_P = int(_rn[0])
_A = int(_rn[1])
_B = int(_rn[2])
_R = int(_rn[3])

_C1024R = (1024 * _R) % _P
_INV_W = np.float32(1.0 / _WSIZE)

_mesh = plsc.VectorSubcoreMesh(core_axis_name="c", subcore_axis_name="s")


@functools.partial(
    pl.kernel,
    out_type=jax.ShapeDtypeStruct((_BATCH, _EMBED_DIM), jnp.float32),
    mesh=_mesh,
    scratch_types=[
        pltpu.VMEM((_ROWS,), jnp.int32),
        pltpu.VMEM((1024,), jnp.int32),
        pltpu.VMEM((1024,), jnp.int32),
        pltpu.VMEM((_ROWS,), jnp.int32),
        pltpu.VMEM((_EMBED_DIM,), jnp.int32),
        pltpu.VMEM((_ROWS * _EMBED_DIM,), jnp.int32),
        pltpu.VMEM((_ROWS, _EMBED_DIM), jnp.float32),
        pltpu.SemaphoreType.DMA,
    ],
    compiler_params=pltpu.CompilerParams(needs_layout_passes=False),
)
def _hashed_lookup(table_h, idx_h, out_h,
                   idx_v, thi_v, tlo_v, base_v, off_v, hidx_v,
                   gath_v, sem):
    wid = lax.axis_index("s") * jnp.int32(_NC) + lax.axis_index("c")
    row0 = wid * jnp.int32(_ROWS)
    pltpu.sync_copy(idx_h.at[pl.ds(row0, _ROWS)], idx_v)

    pconst = jnp.int32(_P)
    wconst = jnp.int32(_WSIZE)
    iota = lax.iota(jnp.int32, 16)
    pmi = pconst - iota

    def _mulmod_iota(c):
        res = jnp.zeros((16,), jnp.int32)
        for bit in bin(c)[2:]:
            t = res - (pconst - res)
            res = jnp.where(t >= 0, t, res + res)
            if bit == "1":
                t = res - pmi
                res = jnp.where(t >= 0, t, res + iota)
        return res

    def _fill(ref, n_groups, v0, step):
        pms = jnp.int32(_P - step)
        sconst = jnp.int32(step)

        def body(g, v):
            ref[pl.ds(g * jnp.int32(16), 16)] = v
            t = v - pms
            return jnp.where(t >= 0, t, v + sconst)

        lax.fori_loop(jnp.int32(0), jnp.int32(n_groups), body, v0)

    v_hi = _mulmod_iota(_C1024R)
    t0 = v_hi - jnp.int32(_P - _A)
    v_hi = jnp.where(t0 >= 0, t0, v_hi + jnp.int32(_A))
    _fill(thi_v, 64, v_hi, (16 * _C1024R) % _P)
    _fill(tlo_v, 64, _mulmod_iota(_R), (16 * _R) % _P)
    _fill(off_v, _EMBED_DIM // 16, _mulmod_iota(_B), (16 * _B) % _P)

    offs = [off_v[pl.ds(k * 16, 16)] for k in range(_EMBED_DIM // 16)]
    pmos = [pconst - o for o in offs]

    def phase_b(g, carry):
        gbase = g * jnp.int32(16 * _EMBED_DIM)
        gsel = g * jnp.int32(16)
        v = idx_v[pl.ds(gsel, 16)]
        hi = lax.shift_right_logical(v, jnp.int32(10))
        lo = lax.bitwise_and(v, jnp.int32(1023))
        a = plsc.load_gather(thi_v, [hi])
        b = plsc.load_gather(tlo_v, [lo])
        dd = a - (pconst - b)
        base_v[pl.ds(gsel, 16)] = jnp.where(dd >= 0, dd, a + b)
        for r in range(16):
            sel = gsel + jnp.full((16,), r, dtype=jnp.int32)
            bi = plsc.load_gather(base_v, [sel])
            for k in range(_EMBED_DIM // 16):
                d = bi - pmos[k]
                t = jnp.where(d >= 0, d, bi + offs[k])
                q = (t.astype(jnp.float32) * _INV_W).astype(jnp.int32)
                rem = t - q * wconst
                rem = jnp.where(rem < 0, rem + wconst, rem)
                rem = jnp.where(rem >= wconst, rem - wconst, rem)
                hidx_v[pl.ds(gbase + jnp.int32(r * _EMBED_DIM + k * 16), 16)] = rem
        return carry

    def block_fn(blk, carry):
        g0 = blk * jnp.int32(_BLKGRP)

        def grp(g2, c2):
            return phase_b(g0 + g2, c2)

        lax.fori_loop(jnp.int32(0), jnp.int32(_BLKGRP), grp, carry)
        cr0 = blk * jnp.int32(_BLKGRP * 16)

        def fire(rr, c2):
            pltpu.async_copy(
                table_h.at[hidx_v.at[pl.ds(rr * jnp.int32(_EMBED_DIM),
                                           _EMBED_DIM)]],
                gath_v.at[rr],
                sem,
            )
            return c2

        lax.fori_loop(cr0, cr0 + jnp.int32(_BLKGRP * 16), fire, carry)
        return carry

    lax.fori_loop(jnp.int32(0), jnp.int32(_GROUPS // _BLKGRP), block_fn,
                  jnp.int32(0))

    def drain(rr, c2):
        pltpu.make_async_copy(
            table_h.at[pl.ds(0, _EMBED_DIM)],
            gath_v.at[rr],
            sem,
        ).wait()
        return c2

    lax.fori_loop(jnp.int32(0), jnp.int32(_ROWS), drain, jnp.int32(0))

    pltpu.sync_copy(gath_v, out_h.at[pl.ds(row0, _ROWS)])


def kernel(hashed_weights, indices):
    idx32 = indices.astype(jnp.int32)
    return _hashed_lookup(hashed_weights, idx32)
